# Initial kernel scaffold; baseline (speedup 1.0000x reference)
import jax
import jax.numpy as jnp
from jax.experimental import pallas as pl


def kernel(q, k, v):
    raise NotImplementedError("write your pallas kernel here")


# speedup vs baseline: 2.4294x; 2.4294x over previous
import jax
import jax.numpy as jnp
from jax.experimental import pallas as pl
from jax.experimental.pallas import tpu as pltpu

BLK = 128
NB = 16
KEEP = 2
NEG = -1e9
FMIN = -3.0e38


def _mask_kernel(q_ref, k_ref, idx_ref):
    q = q_ref[0]
    k = k_ref[0]
    S, D = q.shape
    scale = jnp.float32(1.0) / jnp.sqrt(jnp.float32(D))
    rows = jax.lax.broadcasted_iota(jnp.int32, (NB, S), 0)
    cols = jax.lax.broadcasted_iota(jnp.int32, (NB, S), 1)
    P = jnp.where(cols // BLK == rows, jnp.float32(1.0 / BLK), jnp.float32(0.0))
    qp = jnp.dot(P, q, preferred_element_type=jnp.float32)
    kp = jnp.dot(P, k, preferred_element_type=jnp.float32)
    s = jnp.dot(qp, kp.T, preferred_element_type=jnp.float32) * scale
    col = jax.lax.broadcasted_iota(jnp.int32, (NB, NB), 1)
    m1 = jnp.max(s, axis=1, keepdims=True)
    a1 = jnp.min(jnp.where(s >= m1, col, NB), axis=1)
    s2 = jnp.where(col == a1[:, None], FMIN, s)
    m2 = jnp.max(s2, axis=1, keepdims=True)
    a2 = jnp.min(jnp.where(s2 >= m2, col, NB), axis=1)
    idx_ref[0] = jnp.stack([a1, a2], axis=0)


def _attn_kernel(idx_ref, q_ref, k_ref, v_ref, o_ref):
    h = pl.program_id(0)
    qb = pl.program_id(1)
    i0 = idx_ref[h, 0, qb]
    i1 = idx_ref[h, 1, qb]
    q = q_ref[0]
    scale = jnp.float32(0.125)
    k0 = k_ref[0, pl.ds(i0 * BLK, BLK), :]
    k1 = k_ref[0, pl.ds(i1 * BLK, BLK), :]
    kd = k_ref[0, pl.ds(qb * BLK, BLK), :]
    s0 = jnp.dot(q, k0.T, preferred_element_type=jnp.float32) * scale
    s1 = jnp.dot(q, k1.T, preferred_element_type=jnp.float32) * scale
    sd = jnp.dot(q, kd.T, preferred_element_type=jnp.float32) * scale
    dup = jnp.logical_or(i0 == qb, i1 == qb)
    sd = jnp.where(dup, NEG, sd)
    m = jnp.maximum(jnp.maximum(jnp.max(s0, axis=1), jnp.max(s1, axis=1)),
                    jnp.max(sd, axis=1))[:, None]
    p0 = jnp.exp(s0 - m)
    p1 = jnp.exp(s1 - m)
    pd = jnp.exp(sd - m)
    denom = (jnp.sum(p0, axis=1) + jnp.sum(p1, axis=1)
             + jnp.sum(pd, axis=1))[:, None]
    v0 = v_ref[0, pl.ds(i0 * BLK, BLK), :]
    v1 = v_ref[0, pl.ds(i1 * BLK, BLK), :]
    vd = v_ref[0, pl.ds(qb * BLK, BLK), :]
    acc = jnp.dot(p0, v0, preferred_element_type=jnp.float32)
    acc = acc + jnp.dot(p1, v1, preferred_element_type=jnp.float32)
    acc = acc + jnp.dot(pd, vd, preferred_element_type=jnp.float32)
    o_ref[0] = acc / denom


def kernel(q, k, v):
    B, H, S, D = q.shape
    q3 = q.reshape(B * H, S, D)
    k3 = k.reshape(B * H, S, D)
    v3 = v.reshape(B * H, S, D)
    HH = B * H

    idx = pl.pallas_call(
        _mask_kernel,
        grid=(HH,),
        in_specs=[
            pl.BlockSpec((1, S, D), lambda h: (h, 0, 0)),
            pl.BlockSpec((1, S, D), lambda h: (h, 0, 0)),
        ],
        out_specs=pl.BlockSpec((1, 2, NB), lambda h: (h, 0, 0)),
        out_shape=jax.ShapeDtypeStruct((HH, 2, NB), jnp.int32),
    )(q3, k3)

    out = pl.pallas_call(
        _attn_kernel,
        grid_spec=pltpu.PrefetchScalarGridSpec(
            num_scalar_prefetch=1,
            grid=(HH, NB),
            in_specs=[
                pl.BlockSpec((1, BLK, D), lambda h, qb, idx_ref: (h, qb, 0)),
                pl.BlockSpec((1, S, D), lambda h, qb, idx_ref: (h, 0, 0)),
                pl.BlockSpec((1, S, D), lambda h, qb, idx_ref: (h, 0, 0)),
            ],
            out_specs=pl.BlockSpec((1, BLK, D), lambda h, qb, idx_ref: (h, qb, 0)),
        ),
        out_shape=jax.ShapeDtypeStruct((HH, S, D), jnp.float32),
    )(idx, q3, k3, v3)

    return out.reshape(B, H, S, D)
